# baseline (speedup 1.0000x reference)
import functools

import jax
import jax.numpy as jnp
from jax import lax
from jax.experimental import pallas as pl
from jax.experimental.pallas import tpu as pltpu
from jax.experimental.pallas import tpu_sc as plsc

VOCAB_ = 100000
EMB_ = 512
HID_ = 512
NP_ = HID_ // 128
ENDS_ = (0, 20000, 60000, 100000)
ROWS_PER_BLOCK = 800
SCALE_ = float(HID_) ** 0.5


def _table_body(emb0, emb1, emb2, p0, p1, p2, *outs):
    g = pl.program_id(0)
    nb0 = (ENDS_[1] - ENDS_[0]) // ROWS_PER_BLOCK
    nb1 = (ENDS_[2] - ENDS_[1]) // ROWS_PER_BLOCK

    def store(eref, pref):
        y = lax.dot_general(
            eref[...], pref[...], (((1,), (1,)), ((), ())),
            preferred_element_type=jnp.float32,
        ) * SCALE_
        for k in range(NP_):
            outs[k][...] = y[:, k * 128:(k + 1) * 128]

    @pl.when(g < nb0)
    def _():
        store(emb0, p0)

    @pl.when((g >= nb0) & (g < nb0 + nb1))
    def _():
        store(emb1, p1)

    @pl.when(g >= nb0 + nb1)
    def _():
        store(emb2, p2)


def _build_table(emb_0, emb_1, emb_2, proj_0, proj_1, proj_2):
    r = ROWS_PER_BLOCK
    nb0 = (ENDS_[1] - ENDS_[0]) // r
    nb1 = (ENDS_[2] - ENDS_[1]) // r
    nb2 = (ENDS_[3] - ENDS_[2]) // r
    grid = nb0 + nb1 + nb2
    return pl.pallas_call(
        _table_body,
        grid=(grid,),
        in_specs=[
            pl.BlockSpec((r, EMB_), lambda g: (jnp.minimum(g, nb0 - 1), 0)),
            pl.BlockSpec((r, EMB_ // 2),
                         lambda g: (jnp.clip(g - nb0, 0, nb1 - 1), 0)),
            pl.BlockSpec((r, EMB_ // 4),
                         lambda g: (jnp.clip(g - nb0 - nb1, 0, nb2 - 1), 0)),
            pl.BlockSpec((HID_, EMB_), lambda g: (0, 0)),
            pl.BlockSpec((HID_, EMB_ // 2), lambda g: (0, 0)),
            pl.BlockSpec((HID_, EMB_ // 4), lambda g: (0, 0)),
        ],
        out_specs=[pl.BlockSpec((r, 128), lambda g: (g, 0))
                   for _ in range(NP_)],
        out_shape=[jax.ShapeDtypeStruct((VOCAB_, 128), jnp.float32)
                   for _ in range(NP_)],
    )(emb_0, emb_1, emb_2, proj_0, proj_1, proj_2)


NSLOT_ = 4


@functools.cache
def _make_gather(n_batch, seq):
    info = plsc.get_sparse_core_info()
    nc, ns = info.num_cores, info.num_subcores
    nw = nc * ns
    stride = (seq + 7) // 8 * 8
    assert n_batch % (NSLOT_ * nw) == 0
    b_per_w = n_batch // nw
    mesh = plsc.VectorSubcoreMesh(core_axis_name="c", subcore_axis_name="s")

    @functools.partial(
        pl.kernel,
        mesh=mesh,
        out_type=jax.ShapeDtypeStruct((n_batch, seq, HID_), jnp.float32),
        compiler_params=pltpu.CompilerParams(use_tc_tiling_on_sc=False),
        scratch_types=[
            pltpu.VMEM((b_per_w * stride,), jnp.int32),
            pltpu.VMEM((NSLOT_ * NP_, seq, 128), jnp.float32),
        ] + [pltpu.SemaphoreType.DMA] * (2 * NSLOT_),
    )
    def gather(t0, t1, t2, t3, idx_hbm, out_hbm, idx_v, bufs, *sems):
        gsem = sems[:NSLOT_]
        wsem = sems[NSLOT_:]
        tbls = (t0, t1, t2, t3)
        wid = lax.axis_index("s") * nc + lax.axis_index("c")
        b_base = wid * b_per_w
        pltpu.sync_copy(
            idx_hbm.at[pl.ds(b_base * stride, b_per_w * stride)], idx_v)

        def fire(b, j):
            for k in range(NP_):
                pltpu.async_copy(
                    tbls[k].at[idx_v.at[pl.ds(b * stride, seq)]],
                    bufs.at[NP_ * j + k], gsem[j])

        def drain(sem, j):
            for k in range(NP_):
                pltpu.make_async_copy(
                    tbls[0].at[pl.ds(0, seq)], bufs.at[NP_ * j + k], sem
                ).wait()

        fire(0, 0)
        fire(1, 1)

        def body(i, _):
            for j in range(NSLOT_):
                b = i * NSLOT_ + j
                drain(gsem[j], j)
                for k in range(NP_):
                    pltpu.async_copy(
                        bufs.at[NP_ * j + k],
                        out_hbm.at[b_base + b, :, pl.ds(k * 128, 128)],
                        wsem[j])
                j2 = (j + 2) % NSLOT_
                bt = b + 2

                @pl.when(bt < b_per_w)
                def _():
                    @pl.when(b >= 2)
                    def _():
                        drain(wsem[j2], j2)

                    fire(bt, j2)
            return ()

        lax.fori_loop(0, b_per_w // NSLOT_, body, (), unroll=False)
        for j in range(NSLOT_):
            drain(wsem[j], j)

    return gather


def kernel(token_ids, emb_0, emb_1, emb_2, proj_0, proj_1, proj_2):
    tbls = _build_table(emb_0, emb_1, emb_2, proj_0, proj_1, proj_2)
    n_batch, seq = token_ids.shape
    stride = (seq + 7) // 8 * 8
    ids = jnp.pad(token_ids.astype(jnp.int32), ((0, 0), (0, stride - seq)))
    return _make_gather(n_batch, seq)(*tbls, ids.reshape(-1))


# speedup vs baseline: 1.7396x; 1.7396x over previous
import functools

import jax
import jax.numpy as jnp
from jax import lax
from jax.experimental import pallas as pl
from jax.experimental.pallas import tpu as pltpu
from jax.experimental.pallas import tpu_sc as plsc

VOCAB_ = 100000
EMB_ = 512
HID_ = 512
ENDS_ = (0, 20000, 60000, 100000)
ROWS_PER_BLOCK = 800
SCALE_ = float(HID_) ** 0.5


def _table_body(emb0, emb1, emb2, p0, p1, p2, out):
    g = pl.program_id(0)
    nb0 = (ENDS_[1] - ENDS_[0]) // ROWS_PER_BLOCK
    nb1 = (ENDS_[2] - ENDS_[1]) // ROWS_PER_BLOCK

    def proj(eref, pref):
        return lax.dot_general(
            eref[...], pref[...], (((1,), (1,)), ((), ())),
            preferred_element_type=jnp.float32,
        ) * SCALE_

    @pl.when(g < nb0)
    def _():
        out[...] = proj(emb0, p0)

    @pl.when((g >= nb0) & (g < nb0 + nb1))
    def _():
        out[...] = proj(emb1, p1)

    @pl.when(g >= nb0 + nb1)
    def _():
        out[...] = proj(emb2, p2)


def _build_table(emb_0, emb_1, emb_2, proj_0, proj_1, proj_2):
    r = ROWS_PER_BLOCK
    nb0 = (ENDS_[1] - ENDS_[0]) // r
    nb1 = (ENDS_[2] - ENDS_[1]) // r
    nb2 = (ENDS_[3] - ENDS_[2]) // r
    grid = nb0 + nb1 + nb2
    return pl.pallas_call(
        _table_body,
        grid=(grid,),
        in_specs=[
            pl.BlockSpec((r, EMB_), lambda g: (jnp.minimum(g, nb0 - 1), 0)),
            pl.BlockSpec((r, EMB_ // 2),
                         lambda g: (jnp.clip(g - nb0, 0, nb1 - 1), 0)),
            pl.BlockSpec((r, EMB_ // 4),
                         lambda g: (jnp.clip(g - nb0 - nb1, 0, nb2 - 1), 0)),
            pl.BlockSpec((HID_, EMB_), lambda g: (0, 0)),
            pl.BlockSpec((HID_, EMB_ // 2), lambda g: (0, 0)),
            pl.BlockSpec((HID_, EMB_ // 4), lambda g: (0, 0)),
        ],
        out_specs=pl.BlockSpec((r, HID_), lambda g: (g, 0)),
        out_shape=jax.ShapeDtypeStruct((VOCAB_, HID_), jnp.float32),
    )(emb_0, emb_1, emb_2, proj_0, proj_1, proj_2)


@functools.cache
def _make_gather(seq, n_batch):
    info = plsc.get_sparse_core_info()
    nc, ns = info.num_cores, info.num_subcores
    nw = nc * ns
    b_total = seq * n_batch
    chunk = 64
    assert b_total % (nw * chunk) == 0 and n_batch % chunk == 0
    b_per_w = b_total // nw
    n_chunks = b_per_w // chunk
    mesh = plsc.VectorSubcoreMesh(core_axis_name="c", subcore_axis_name="s")

    @functools.partial(
        pl.kernel,
        mesh=mesh,
        out_type=jax.ShapeDtypeStruct((seq, n_batch, HID_), jnp.float32),
        scratch_types=[
            pltpu.VMEM((b_per_w,), jnp.int32),
            pltpu.VMEM((chunk, HID_), jnp.float32),
            pltpu.VMEM((chunk, HID_), jnp.float32),
            pltpu.SemaphoreType.DMA,
            pltpu.SemaphoreType.DMA,
        ],
    )
    def gather(table_hbm, idx_hbm, out_hbm, idx_v, rows_a, rows_b, sem_a,
               sem_b):
        wid = lax.axis_index("s") * nc + lax.axis_index("c")
        base = wid * b_per_w
        pltpu.sync_copy(idx_hbm.at[pl.ds(base, b_per_w)], idx_v)
        bufs = (rows_a, rows_b)
        sems = (sem_a, sem_b)
        copies = [None, None]
        copies[0] = pltpu.async_copy(
            table_hbm.at[idx_v.at[pl.ds(0, chunk)]], bufs[0], sems[0])
        for c in range(n_chunks):
            if c + 1 < n_chunks:
                copies[(c + 1) % 2] = pltpu.async_copy(
                    table_hbm.at[idx_v.at[pl.ds((c + 1) * chunk, chunk)]],
                    bufs[(c + 1) % 2], sems[(c + 1) % 2])
            copies[c % 2].wait()
            flat = base + c * chunk
            s = flat // n_batch
            b0 = flat % n_batch
            pltpu.sync_copy(bufs[c % 2], out_hbm.at[s, pl.ds(b0, chunk)])

    return gather


def kernel(token_ids, emb_0, emb_1, emb_2, proj_0, proj_1, proj_2):
    table = _build_table(emb_0, emb_1, emb_2, proj_0, proj_1, proj_2)
    n_batch, seq = token_ids.shape
    idx = token_ids.astype(jnp.int32).T.reshape(-1)
    out_sm = _make_gather(seq, n_batch)(table, idx)
    return jnp.transpose(out_sm, (1, 0, 2))
